# baseline (device time: 46785 ns/iter reference)
import jax
import jax.numpy as jnp
from jax import lax
from jax.experimental import pallas as pl
from jax.experimental.pallas import tpu as pltpu

N_DEV = 4

SEED_A, DIR_B, RELAY_A, SEED_B, DIR_A, RELAY_B = range(6)
_TO_RIGHT = {SEED_A, DIR_B, RELAY_A}


def kernel(x, w_mat):
    k, n = w_mat.shape
    m = x.shape[0]
    m_per = m // N_DEV
    n_half = n // 2

    def body(x_ref, w_ref, out_ref, sbuf, rbuf, ss, rs):
        my = lax.axis_index("i")
        left = lax.rem(my + (N_DEV - 1), N_DEV)
        right = lax.rem(my + 1, N_DEV)

        barrier_sem = pltpu.get_barrier_semaphore()
        for nbr in (left, right):
            pl.semaphore_signal(
                barrier_sem, inc=1,
                device_id=(nbr,), device_id_type=pl.DeviceIdType.MESH,
            )
        pl.semaphore_wait(barrier_sem, 2)

        w = w_ref[...].astype(jnp.bfloat16)

        def partial(c, lo):
            xs = x_ref[pl.ds(c * m_per, m_per), :].astype(jnp.bfloat16)
            return jnp.dot(
                xs, w[:, lo:lo + n_half], preferred_element_type=jnp.float32
            )

        c_dm1 = lax.rem(my + N_DEV - 1, N_DEV)
        c_dp1 = lax.rem(my + 1, N_DEV)
        c_dp2 = lax.rem(my + 2, N_DEV)

        def mk(slot):
            return pltpu.make_async_remote_copy(
                src_ref=sbuf.at[slot], dst_ref=rbuf.at[slot],
                send_sem=ss.at[slot], recv_sem=rs.at[slot],
                device_id=(right if slot in _TO_RIGHT else left,),
                device_id_type=pl.DeviceIdType.MESH,
            )

        sbuf[SEED_A] = partial(c_dp2, 0).astype(jnp.bfloat16)
        mk(SEED_A).start()
        sbuf[SEED_B] = partial(c_dp2, n_half).astype(jnp.bfloat16)
        mk(SEED_B).start()
        sbuf[DIR_B] = partial(c_dp1, n_half).astype(jnp.bfloat16)
        mk(DIR_B).start()
        sbuf[DIR_A] = partial(c_dm1, 0).astype(jnp.bfloat16)
        mk(DIR_A).start()

        p_relay_a = partial(c_dp1, 0)
        p_relay_b = partial(c_dm1, n_half)
        pd_a = partial(my, 0)
        pd_b = partial(my, n_half)

        mk(SEED_A).wait_recv()
        sbuf[RELAY_A] = (
            rbuf[SEED_A].astype(jnp.float32) + p_relay_a
        ).astype(jnp.bfloat16)
        mk(RELAY_A).start()
        mk(SEED_B).wait_recv()
        sbuf[RELAY_B] = (
            rbuf[SEED_B].astype(jnp.float32) + p_relay_b
        ).astype(jnp.bfloat16)
        mk(RELAY_B).start()

        mk(DIR_A).wait_recv()
        pre_a = rbuf[DIR_A].astype(jnp.float32) + pd_a
        mk(DIR_B).wait_recv()
        pre_b = rbuf[DIR_B].astype(jnp.float32) + pd_b

        mk(RELAY_A).wait_recv()
        out_ref[:, :n_half] = jnp.maximum(
            rbuf[RELAY_A].astype(jnp.float32) + pre_a, 0.0,
        )
        mk(RELAY_B).wait_recv()
        out_ref[:, n_half:] = jnp.maximum(
            rbuf[RELAY_B].astype(jnp.float32) + pre_b, 0.0,
        )

        for slot in range(6):
            mk(slot).wait_send()

    comm = pltpu.VMEM((6, m_per, n_half), jnp.bfloat16)
    sems = pltpu.SemaphoreType.DMA((6,))
    return pl.pallas_call(
        body,
        out_shape=jax.ShapeDtypeStruct((m_per, n), jnp.float32),
        in_specs=[
            pl.BlockSpec(memory_space=pltpu.VMEM),
            pl.BlockSpec(memory_space=pltpu.VMEM),
        ],
        out_specs=pl.BlockSpec(memory_space=pltpu.VMEM),
        scratch_shapes=[comm, comm, sems, sems],
        compiler_params=pltpu.CompilerParams(collective_id=0),
    )(x, w_mat)
